# Initial kernel scaffold; baseline (speedup 1.0000x reference)
import jax
import jax.numpy as jnp
from jax.experimental import pallas as pl


def kernel(x, edge_index, batch, W1l, W1r, b1, W2l, W2r, b2, Wfc, bfc):
    raise NotImplementedError("write your pallas kernel here")


# speedup vs baseline: 4.1979x; 4.1979x over previous
import functools

import jax
import jax.numpy as jnp
from jax import lax
from jax.experimental import pallas as pl
from jax.experimental.pallas import tpu as pltpu
from jax.experimental.pallas import tpu_sc as plsc

NC, NS = 2, 16
NW = NC * NS
E_CHUNK = 80
RC = 80
ROW_BLK = 1000
N_GR = 64


def _stripes(n_nodes):
    full = ((n_nodes // NS) + RC - 1) // RC * RC
    tail = n_nodes - (NS - 1) * full
    assert tail > 0 and tail % RC == 0 and full % RC == 0
    return full, full // RC, tail // RC


def _make_sc_agg1(n_nodes, n_edges, d):
    full, fullc, tailc = _stripes(n_nodes)
    epw = n_edges // NW
    mesh = plsc.VectorSubcoreMesh(core_axis_name="c", subcore_axis_name="s",
                                  num_cores=NC, num_subcores=NS)

    @functools.partial(
        pl.kernel,
        out_type=(
            jax.ShapeDtypeStruct((NC, n_nodes, d), jnp.float32),
            jax.ShapeDtypeStruct((NC, n_nodes, d), jnp.float32),
        ),
        mesh=mesh,
        scratch_types=[
            pltpu.VMEM((E_CHUNK,), jnp.int32),
            pltpu.VMEM((E_CHUNK,), jnp.int32),
            pltpu.VMEM((E_CHUNK, d), jnp.float32),
            pltpu.VMEM((E_CHUNK, d), jnp.float32),
            pltpu.VMEM_SHARED((n_nodes, d), jnp.float32),
            pltpu.SemaphoreType.DMA,
        ],
    )
    def sc_agg1(x_hbm, src_hbm, dst_hbm, zd_hbm, ones_hbm,
                agg_out, deg_out,
                idx_s, idx_d, rows, ones_v, acc_sh, sem):
        c = lax.axis_index("c")
        s = lax.axis_index("s")
        wid = s * NC + c
        nchunks = jnp.where(s < NS - 1, fullc, tailc)
        rbase = s * full
        ebase = wid * epw

        def zero_acc():
            def zstep(j, carry):
                pltpu.sync_copy(rows, acc_sh.at[pl.ds(rbase + j * RC, RC)])
                return carry

            lax.fori_loop(0, nchunks, zstep, 0)

        def publish(out_hbm):
            def pstep(j, carry):
                sl = pl.ds(rbase + j * RC, RC)
                pltpu.sync_copy(acc_sh.at[sl], rows)
                pltpu.sync_copy(rows, out_hbm.at[c, sl])
                return carry

            lax.fori_loop(0, nchunks, pstep, 0)

        pltpu.sync_copy(zd_hbm, rows)
        pltpu.sync_copy(ones_hbm, ones_v)
        zero_acc()
        plsc.subcore_barrier()

        def dstep(j, carry):
            off = ebase + j * E_CHUNK
            pltpu.sync_copy(dst_hbm.at[pl.ds(off, E_CHUNK)], idx_d)
            pltpu.sync_copy(ones_v, acc_sh.at[idx_d], add=True)
            return carry

        lax.fori_loop(0, epw // E_CHUNK, dstep, 0)
        plsc.subcore_barrier()
        publish(deg_out)
        plsc.subcore_barrier()

        pltpu.sync_copy(zd_hbm, rows)
        zero_acc()
        plsc.subcore_barrier()

        def astep(j, carry):
            off = ebase + j * E_CHUNK
            pltpu.sync_copy(src_hbm.at[pl.ds(off, E_CHUNK)], idx_s)
            pltpu.sync_copy(dst_hbm.at[pl.ds(off, E_CHUNK)], idx_d)
            pltpu.async_copy(x_hbm.at[idx_s], rows, sem).wait()
            pltpu.sync_copy(rows, acc_sh.at[idx_d], add=True)
            return carry

        lax.fori_loop(0, epw // E_CHUNK, astep, 0)
        plsc.subcore_barrier()
        publish(agg_out)

    return sc_agg1


def _make_sc_agg2(n_nodes, n_edges, d):
    full, fullc, tailc = _stripes(n_nodes)
    eps = n_edges // NS
    mesh = plsc.VectorSubcoreMesh(core_axis_name="c", subcore_axis_name="s",
                                  num_cores=NC, num_subcores=NS)

    @functools.partial(
        pl.kernel,
        out_type=jax.ShapeDtypeStruct((NC, n_nodes, d), jnp.float32),
        mesh=mesh,
        scratch_types=[
            pltpu.VMEM((E_CHUNK,), jnp.int32),
            pltpu.VMEM((E_CHUNK,), jnp.int32),
            pltpu.VMEM((E_CHUNK, d), jnp.float32),
            pltpu.VMEM_SHARED((n_nodes, d), jnp.float32),
            pltpu.SemaphoreType.DMA,
        ],
    )
    def sc_agg2(hlo_hbm, hhi_hbm, src_hbm, dst_hbm, zd_hbm,
                agg_out,
                idx_s, idx_d, rows, acc_sh, sem):
        c = lax.axis_index("c")
        s = lax.axis_index("s")
        nchunks = jnp.where(s < NS - 1, fullc, tailc)
        rbase = s * full

        pltpu.sync_copy(zd_hbm, rows)

        def zstep(j, carry):
            pltpu.sync_copy(rows, acc_sh.at[pl.ds(rbase + j * RC, RC)])
            return carry

        lax.fori_loop(0, nchunks, zstep, 0)
        plsc.subcore_barrier()

        ebase = s * eps

        def run(table_hbm):
            def step(j, carry):
                off = ebase + j * E_CHUNK
                pltpu.sync_copy(src_hbm.at[pl.ds(off, E_CHUNK)], idx_s)
                pltpu.sync_copy(dst_hbm.at[pl.ds(off, E_CHUNK)], idx_d)
                pltpu.async_copy(table_hbm.at[idx_s], rows, sem).wait()
                pltpu.sync_copy(rows, acc_sh.at[idx_d], add=True)
                return carry

            lax.fori_loop(0, eps // E_CHUNK, step, 0)

        @pl.when(c == 0)
        def _():
            run(hlo_hbm)

        @pl.when(c == 1)
        def _():
            run(hhi_hbm)

        plsc.subcore_barrier()

        def pstep(j, carry):
            sl = pl.ds(rbase + j * RC, RC)
            pltpu.sync_copy(acc_sh.at[sl], rows)
            pltpu.sync_copy(rows, agg_out.at[c, sl])
            return carry

        lax.fori_loop(0, nchunks, pstep, 0)

    return sc_agg2


def _tc1_body(agg_ref, deg_ref, x_ref, w1l_ref, w1r_ref, b1_ref,
              lo_ref, hi_ref):
    deg = jnp.maximum(deg_ref[0, :, 0:1] + deg_ref[1, :, 0:1], 1.0)
    agg = (agg_ref[0] + agg_ref[1]) / deg
    h = jnp.dot(agg, w1l_ref[...], preferred_element_type=jnp.float32)
    h += jnp.dot(x_ref[...], w1r_ref[...], preferred_element_type=jnp.float32)
    h = jnp.maximum(h + b1_ref[...], 0.0)
    d = h.shape[1] // 2
    lo_ref[...] = h[:, :d]
    hi_ref[...] = h[:, d:]


def _tc1(agg1, deg, x, W1l, W1r, b1):
    n, din = x.shape
    dh = W1l.shape[1]
    grid = (n // ROW_BLK,)
    return pl.pallas_call(
        _tc1_body,
        grid=grid,
        in_specs=[
            pl.BlockSpec((NC, ROW_BLK, din), lambda i: (0, i, 0)),
            pl.BlockSpec((NC, ROW_BLK, din), lambda i: (0, i, 0)),
            pl.BlockSpec((ROW_BLK, din), lambda i: (i, 0)),
            pl.BlockSpec((din, dh), lambda i: (0, 0)),
            pl.BlockSpec((din, dh), lambda i: (0, 0)),
            pl.BlockSpec((1, dh), lambda i: (0, 0)),
        ],
        out_specs=[
            pl.BlockSpec((ROW_BLK, dh // 2), lambda i: (i, 0)),
            pl.BlockSpec((ROW_BLK, dh // 2), lambda i: (i, 0)),
        ],
        out_shape=[
            jax.ShapeDtypeStruct((n, dh // 2), jnp.float32),
            jax.ShapeDtypeStruct((n, dh // 2), jnp.float32),
        ],
    )(agg1, deg, x, W1l, W1r, b1)


def _tc2_body(agg_ref, deg_ref, lo_ref, hi_ref, batch_ref,
              w2l_ref, w2r_ref, b2_ref, wfc_ref, bfc_ref,
              out_ref, pooled_acc, cnt_acc):
    i = pl.program_id(0)
    dhalf = agg_ref.shape[2]
    deg = jnp.maximum(deg_ref[0, :, 0:1] + deg_ref[1, :, 0:1], 1.0)
    h = jnp.dot(agg_ref[0] / deg, w2l_ref[:dhalf, :],
                preferred_element_type=jnp.float32)
    h += jnp.dot(agg_ref[1] / deg, w2l_ref[dhalf:, :],
                 preferred_element_type=jnp.float32)
    h += jnp.dot(lo_ref[...], w2r_ref[:dhalf, :],
                 preferred_element_type=jnp.float32)
    h += jnp.dot(hi_ref[...], w2r_ref[dhalf:, :],
                 preferred_element_type=jnp.float32)
    h = jnp.maximum(h + b2_ref[...], 0.0)

    b = batch_ref[0]
    onehot = (lax.broadcasted_iota(jnp.int32, (N_GR, 1), 0) == b
              ).astype(jnp.float32)
    ps = jnp.dot(onehot, h, preferred_element_type=jnp.float32)
    cs = jnp.sum(onehot, axis=1, keepdims=True)

    @pl.when(i == 0)
    def _():
        pooled_acc[...] = ps
        cnt_acc[...] = cs

    @pl.when(i > 0)
    def _():
        pooled_acc[...] += ps
        cnt_acc[...] += cs

    @pl.when(i == pl.num_programs(0) - 1)
    def _():
        pooled = pooled_acc[...] / jnp.maximum(cnt_acc[...], 1.0)
        out_ref[...] = jnp.dot(pooled, wfc_ref[...],
                               preferred_element_type=jnp.float32) + bfc_ref[...]


def _tc2(agg2, deg, hlo, hhi, batch3, W2l, W2r, b2, Wfc, bfc):
    n = hlo.shape[0]
    dhalf = hlo.shape[1]
    dh = W2l.shape[0]
    ncls = Wfc.shape[1]
    grid = (n // ROW_BLK,)
    return pl.pallas_call(
        _tc2_body,
        grid=grid,
        in_specs=[
            pl.BlockSpec((NC, ROW_BLK, dhalf), lambda i: (0, i, 0)),
            pl.BlockSpec((NC, ROW_BLK, dhalf), lambda i: (0, i, 0)),
            pl.BlockSpec((ROW_BLK, dhalf), lambda i: (i, 0)),
            pl.BlockSpec((ROW_BLK, dhalf), lambda i: (i, 0)),
            pl.BlockSpec((1, 1, ROW_BLK), lambda i: (i, 0, 0)),
            pl.BlockSpec((dh, dh), lambda i: (0, 0)),
            pl.BlockSpec((dh, dh), lambda i: (0, 0)),
            pl.BlockSpec((1, dh), lambda i: (0, 0)),
            pl.BlockSpec((dh, ncls), lambda i: (0, 0)),
            pl.BlockSpec((1, ncls), lambda i: (0, 0)),
        ],
        out_specs=pl.BlockSpec((N_GR, ncls), lambda i: (0, 0)),
        out_shape=jax.ShapeDtypeStruct((N_GR, ncls), jnp.float32),
        scratch_shapes=[
            pltpu.VMEM((N_GR, dh), jnp.float32),
            pltpu.VMEM((N_GR, 1), jnp.float32),
        ],
    )(agg2, deg, hlo, hhi, batch3, W2l, W2r, b2, Wfc, bfc)


def kernel(x, edge_index, batch, W1l, W1r, b1, W2l, W2r, b2, Wfc, bfc):
    n, din = x.shape
    e = edge_index.shape[1]
    dh = W1l.shape[1]
    src = edge_index[0].astype(jnp.int32)
    dst = edge_index[1].astype(jnp.int32)

    z_din = jnp.zeros((E_CHUNK, din), jnp.float32)
    ones_din = jnp.ones((E_CHUNK, din), jnp.float32)
    z_half = jnp.zeros((E_CHUNK, dh // 2), jnp.float32)

    agg1, deg = _make_sc_agg1(n, e, din)(x, src, dst, z_din, ones_din)
    hlo, hhi = _tc1(agg1, deg, x, W1l, W1r, b1.reshape(1, -1))
    agg2 = _make_sc_agg2(n, e, dh // 2)(hlo, hhi, src, dst, z_half)
    batch3 = batch.astype(jnp.int32).reshape(n // ROW_BLK, 1, ROW_BLK)
    return _tc2(agg2, deg, hlo, hhi, batch3,
                W2l, W2r, b2.reshape(1, -1), Wfc, bfc.reshape(1, -1))
